# baseline (speedup 1.0000x reference)
import jax
import jax.numpy as jnp
from jax import lax
from jax.experimental import pallas as pl
from jax.experimental.pallas import tpu as pltpu
from jax.experimental.pallas import tpu_sc as plsc

N = 100000
T = 3200000
NC, NS, L = 2, 16, 16
NW = NC * NS
TW = T // NW
B = 2000
NCHUNK = TW // B


def _tri_kernel(y0_hbm, y1_hbm, w_hbm, ti_hbm, tj_hbm, tk_hbm, out_hbm,
                idx_i0, idx_j0, idx_k0, idx_i1, idx_j1, idx_k1,
                w_v0, w_v1,
                yi00, yi10, yj00, yj10, yk00, yk10,
                yi01, yi11, yj01, yj11, yk01, yk11,
                acc_v, sem_idx, sem_g0, sem_g1):
    wid = lax.axis_index("s") * NC + lax.axis_index("c")
    idx = ((idx_i0, idx_j0, idx_k0), (idx_i1, idx_j1, idx_k1))
    w_v = (w_v0, w_v1)
    vals = (((yi00, yi10), (yj00, yj10), (yk00, yk10)),
            ((yi01, yi11), (yj01, yj11), (yk01, yk11)))
    sems = (sem_g0, sem_g1)
    tabs = (y0_hbm, y1_hbm)

    def stage_and_fire(c, b):
        base = wid * TW + c * B
        cps = [
            pltpu.async_copy(ti_hbm.at[pl.ds(base, B)], idx[b][0], sem_idx),
            pltpu.async_copy(tj_hbm.at[pl.ds(base, B)], idx[b][1], sem_idx),
            pltpu.async_copy(tk_hbm.at[pl.ds(base, B)], idx[b][2], sem_idx),
        ]
        pltpu.async_copy(w_hbm.at[pl.ds(base, B)], w_v[b], sems[b])
        for cp in cps:
            cp.wait()
        for col in range(3):
            for d in range(2):
                pltpu.async_copy(tabs[d].at[idx[b][col]], vals[b][col][d],
                                 sems[b])

    def wait_chunk(b):
        pltpu.make_async_copy(w_hbm.at[pl.ds(0, B)], w_v[b], sems[b]).wait()
        for col in range(3):
            for d in range(2):
                pltpu.make_async_copy(
                    tabs[d].at[idx[b][col]], vals[b][col][d], sems[b]).wait()

    def compute_chunk(b, carry):
        (yi0, yi1), (yj0, yj1), (yk0, yk1) = vals[b]
        wv = w_v[b]

        def lane_body(l, inner):
            la, va = inner
            s = pl.ds(l * L, L)
            a0 = yi0[s]
            a1 = yi1[s]
            dx = a0 - yj0[s]
            dy = a1 - yj1[s]
            ex = a0 - yk0[s]
            ey = a1 - yk1[s]
            d_ij = 1.0 + (dx * dx + dy * dy)
            d_ik = 1.0 + (ex * ex + ey * ey)
            la = la + wv[s] * (d_ij / (d_ij + d_ik))
            va = va + jnp.where(d_ij > d_ik, 1.0, 0.0).astype(jnp.float32)
            return la, va

        return lax.fori_loop(0, B // L, lane_body, carry)

    stage_and_fire(0, 0)
    stage_and_fire(1, 1)

    def pair_body(c2, carry):
        for b in (0, 1):
            c = 2 * c2 + b
            wait_chunk(b)
            carry = compute_chunk(b, carry)

            @pl.when(c + 2 < NCHUNK)
            def _():
                stage_and_fire(c + 2, b)
        return carry

    zero = jnp.zeros((L,), jnp.float32)
    loss_acc, viol_acc = lax.fori_loop(0, NCHUNK // 2, pair_body, (zero, zero))
    acc_v[...] = loss_acc
    pltpu.sync_copy(acc_v, out_hbm.at[0, pl.ds(wid * L, L)])
    acc_v[...] = viol_acc
    pltpu.sync_copy(acc_v, out_hbm.at[1, pl.ds(wid * L, L)])


@jax.jit
def kernel(Y, weights, triplets):
    y0 = Y[:, 0]
    y1 = Y[:, 1]
    ti = triplets[:, 0].astype(jnp.int32)
    tj = triplets[:, 1].astype(jnp.int32)
    tk = triplets[:, 2].astype(jnp.int32)

    mesh = plsc.VectorSubcoreMesh(core_axis_name="c", subcore_axis_name="s")
    run = pl.kernel(
        _tri_kernel,
        out_type=jax.ShapeDtypeStruct((2, NW * L), jnp.float32),
        mesh=mesh,
        scratch_types=(
            [pltpu.VMEM((B,), jnp.int32) for _ in range(6)]
            + [pltpu.VMEM((B,), jnp.float32) for _ in range(2)]
            + [pltpu.VMEM((B,), jnp.float32) for _ in range(12)]
            + [pltpu.VMEM((L,), jnp.float32)]
            + [pltpu.SemaphoreType.DMA] * 3
        ),
    )
    partials = run(y0, y1, weights, ti, tj, tk)
    loss = jnp.sum(partials[0])
    num_viol = jnp.sum(partials[1])
    return (loss, num_viol)


# speedup vs baseline: 3.7068x; 3.7068x over previous
import jax
import jax.numpy as jnp
from jax import lax
from jax.experimental import pallas as pl
from jax.experimental.pallas import tpu as pltpu
from jax.experimental.pallas import tpu_sc as plsc

N = 100000
T = 3200000
NC, NS, L = 2, 16, 16
NW = NC * NS
TW = T // NW
B = 2000
NCHUNK = TW // B


def _tri_kernel(y0_hbm, y1_hbm, w_hbm, ti_hbm, tj_hbm, tk_hbm, out_hbm,
                sh_y0, sh_y1,
                idx_i0, idx_j0, idx_k0, idx_i1, idx_j1, idx_k1,
                w_v0, w_v1,
                yi00, yi10, yj00, yj10, yk00, yk10,
                yi01, yi11, yj01, yj11, yk01, yk11,
                acc_v, sem_idx, sem_g0, sem_g1):
    wid = lax.axis_index("s") * NC + lax.axis_index("c")
    idx = ((idx_i0, idx_j0, idx_k0), (idx_i1, idx_j1, idx_k1))
    w_v = (w_v0, w_v1)
    vals = (((yi00, yi10), (yj00, yj10), (yk00, yk10)),
            ((yi01, yi11), (yj01, yj11), (yk01, yk11)))
    sems = (sem_g0, sem_g1)
    tabs = (sh_y0, sh_y1)

    @pl.when(lax.axis_index("s") == 0)
    def _():
        pltpu.sync_copy(y0_hbm, sh_y0)
        pltpu.sync_copy(y1_hbm, sh_y1)
    plsc.subcore_barrier()

    def stage_and_fire(c, b):
        base = wid * TW + c * B
        cps = [
            pltpu.async_copy(ti_hbm.at[pl.ds(base, B)], idx[b][0], sem_idx),
            pltpu.async_copy(tj_hbm.at[pl.ds(base, B)], idx[b][1], sem_idx),
            pltpu.async_copy(tk_hbm.at[pl.ds(base, B)], idx[b][2], sem_idx),
        ]
        pltpu.async_copy(w_hbm.at[pl.ds(base, B)], w_v[b], sems[b])
        for cp in cps:
            cp.wait()
        for col in range(3):
            for d in range(2):
                pltpu.async_copy(tabs[d].at[idx[b][col]], vals[b][col][d],
                                 sems[b])

    def wait_chunk(b):
        pltpu.make_async_copy(w_hbm.at[pl.ds(0, B)], w_v[b], sems[b]).wait()
        for col in range(3):
            for d in range(2):
                pltpu.make_async_copy(
                    tabs[d].at[idx[b][col]], vals[b][col][d], sems[b]).wait()

    def compute_chunk(b, carry):
        (yi0, yi1), (yj0, yj1), (yk0, yk1) = vals[b]
        wv = w_v[b]

        def lane_body(l, inner):
            la, va = inner
            s = pl.ds(l * L, L)
            a0 = yi0[s]
            a1 = yi1[s]
            dx = a0 - yj0[s]
            dy = a1 - yj1[s]
            ex = a0 - yk0[s]
            ey = a1 - yk1[s]
            d_ij = 1.0 + (dx * dx + dy * dy)
            d_ik = 1.0 + (ex * ex + ey * ey)
            la = la + wv[s] * (d_ij / (d_ij + d_ik))
            va = va + jnp.where(d_ij > d_ik, 1.0, 0.0).astype(jnp.float32)
            return la, va

        return lax.fori_loop(0, B // L, lane_body, carry)

    stage_and_fire(0, 0)
    stage_and_fire(1, 1)

    def pair_body(c2, carry):
        for b in (0, 1):
            c = 2 * c2 + b
            wait_chunk(b)
            carry = compute_chunk(b, carry)

            @pl.when(c + 2 < NCHUNK)
            def _():
                stage_and_fire(c + 2, b)
        return carry

    zero = jnp.zeros((L,), jnp.float32)
    loss_acc, viol_acc = lax.fori_loop(0, NCHUNK // 2, pair_body, (zero, zero))
    acc_v[...] = loss_acc
    pltpu.sync_copy(acc_v, out_hbm.at[0, pl.ds(wid * L, L)])
    acc_v[...] = viol_acc
    pltpu.sync_copy(acc_v, out_hbm.at[1, pl.ds(wid * L, L)])


@jax.jit
def kernel(Y, weights, triplets):
    y0 = Y[:, 0]
    y1 = Y[:, 1]
    ti = triplets[:, 0].astype(jnp.int32)
    tj = triplets[:, 1].astype(jnp.int32)
    tk = triplets[:, 2].astype(jnp.int32)

    mesh = plsc.VectorSubcoreMesh(core_axis_name="c", subcore_axis_name="s")
    run = pl.kernel(
        _tri_kernel,
        out_type=jax.ShapeDtypeStruct((2, NW * L), jnp.float32),
        mesh=mesh,
        scratch_types=(
            [pltpu.VMEM_SHARED((N,), jnp.float32) for _ in range(2)]
            + [pltpu.VMEM((B,), jnp.int32) for _ in range(6)]
            + [pltpu.VMEM((B,), jnp.float32) for _ in range(2)]
            + [pltpu.VMEM((B,), jnp.float32) for _ in range(12)]
            + [pltpu.VMEM((L,), jnp.float32)]
            + [pltpu.SemaphoreType.DMA] * 3
        ),
    )
    partials = run(y0, y1, weights, ti, tj, tk)
    loss = jnp.sum(partials[0])
    num_viol = jnp.sum(partials[1])
    return (loss, num_viol)
